# baseline (speedup 1.0000x reference)
import functools

import jax
import jax.numpy as jnp
from jax import lax
from jax.experimental import pallas as pl
from jax.experimental.pallas import tpu as pltpu
from jax.experimental.pallas import tpu_sc as plsc

NUM_EXPERTS = 16
TOP_K = 2
BLOCK_T = 1024

_INFO = plsc.get_sparse_core_info()
_NC, _NS, _L = _INFO.num_cores, _INFO.num_subcores, _INFO.num_lanes
NW = _NC * _NS


def _matmul_body(x_ref, w_ref, out_ref):
    x = x_ref[...]
    w = w_ref[...]
    logits = lax.dot_general(
        w, x, (((1,), (1,)), ((), ())),
        preferred_element_type=jnp.float32)
    tpw = out_ref.shape[2]
    nblk = BLOCK_T // tpw
    out_ref[...] = logits.reshape(NUM_EXPERTS, nblk, tpw).transpose(1, 0, 2)


def _sc_gate_body(l_hbm, vals_hbm, idx_hbm, tile, vout, iout):
    tpw = tile.shape[1]
    wid = lax.axis_index("s") * _NC + lax.axis_index("c")
    pltpu.sync_copy(l_hbm.at[wid], tile)
    for g in range(tpw // _L):
        sl = pl.ds(g * _L, _L)
        ls = [tile[e, sl] for e in range(NUM_EXPERTS)]
        m = ls[0]
        for e in range(1, NUM_EXPERTS):
            m = jnp.maximum(m, ls[e])
        s = jnp.exp(ls[0] - m)
        for e in range(1, NUM_EXPERTS):
            s = s + jnp.exp(ls[e] - m)
        max1 = ls[0]
        idx1 = jnp.zeros((_L,), jnp.int32)
        max2 = jnp.full((_L,), -jnp.inf, jnp.float32)
        idx2 = jnp.zeros((_L,), jnp.int32)
        for e in range(1, NUM_EXPERTS):
            ev = jnp.full((_L,), e, jnp.int32)
            gt1 = ls[e] > max1
            gt2 = ls[e] > max2
            idx2 = jnp.where(gt1, idx1, jnp.where(gt2, ev, idx2))
            max2 = jnp.where(gt1, max1, jnp.where(gt2, ls[e], max2))
            idx1 = jnp.where(gt1, ev, idx1)
            max1 = jnp.where(gt1, ls[e], max1)
        vout[0, sl] = jnp.exp(max1 - m) / s
        vout[1, sl] = jnp.exp(max2 - m) / s
        iout[0, sl] = idx1
        iout[1, sl] = idx2
    pltpu.sync_copy(vout, vals_hbm.at[wid])
    pltpu.sync_copy(iout, idx_hbm.at[wid])


@jax.jit
def kernel(hidden_states, weight):
    x = hidden_states.reshape(-1, hidden_states.shape[-1])
    t, d = x.shape
    tpw = t // NW
    nblk = BLOCK_T // tpw
    logits = pl.pallas_call(
        _matmul_body,
        grid=(t // BLOCK_T,),
        in_specs=[
            pl.BlockSpec((BLOCK_T, d), lambda i: (i, 0)),
            pl.BlockSpec((NUM_EXPERTS, d), lambda i: (0, 0)),
        ],
        out_specs=pl.BlockSpec((nblk, NUM_EXPERTS, tpw), lambda i: (i, 0, 0)),
        out_shape=jax.ShapeDtypeStruct((NW, NUM_EXPERTS, tpw), jnp.float32),
    )(x, weight)

    sc_gate = functools.partial(
        pl.kernel,
        mesh=plsc.VectorSubcoreMesh(core_axis_name="c", subcore_axis_name="s"),
        out_type=[
            jax.ShapeDtypeStruct((NW, TOP_K, tpw), jnp.float32),
            jax.ShapeDtypeStruct((NW, TOP_K, tpw), jnp.int32),
        ],
        scratch_types=[
            pltpu.VMEM((NUM_EXPERTS, tpw), jnp.float32),
            pltpu.VMEM((TOP_K, tpw), jnp.float32),
            pltpu.VMEM((TOP_K, tpw), jnp.int32),
        ],
    )(_sc_gate_body)
    vals_w, idx_w = sc_gate(logits)
    vals = vals_w.transpose(0, 2, 1).reshape(t, TOP_K)
    idx = idx_w.transpose(0, 2, 1).reshape(t, TOP_K)
    return vals, idx
